# baseline (speedup 1.0000x reference)
import functools

import jax
import jax.numpy as jnp
from jax import lax
from jax.experimental import pallas as pl
from jax.experimental.pallas import tpu as pltpu
from jax.experimental.pallas import tpu_sc as plsc

_NUM_FIELDS = 26
_VOCAB = 100000
_LO = 49920
_HI = _VOCAB - _LO
_EMBED_DIM = 32
_BATCH = 16384
_NPLANE = _NUM_FIELDS * _EMBED_DIM
_BCH = 4096
_NCH = _BATCH // _BCH
_UNROLL = 4
_NIT = _BCH // (16 * _UNROLL)

_mesh = plsc.VectorSubcoreMesh(core_axis_name="c", subcore_axis_name="s")


@functools.partial(
    pl.kernel,
    mesh=_mesh,
    out_type=jax.ShapeDtypeStruct((_NPLANE, _BATCH), jnp.float32),
    scratch_types=[
        pltpu.VMEM((_LO,), jnp.float32),
        pltpu.VMEM((_HI,), jnp.float32),
        pltpu.VMEM((_BCH,), jnp.int32),
        pltpu.VMEM((_BCH,), jnp.int32),
        pltpu.VMEM((_BCH,), jnp.float32),
        pltpu.VMEM((_BCH,), jnp.float32),
        pltpu.VMEM((_BCH,), jnp.float32),
        pltpu.VMEM((_BCH,), jnp.float32),
        pltpu.VMEM_SHARED((2, _BATCH), jnp.int32),
        pltpu.SemaphoreType.DMA,
        pltpu.SemaphoreType.DMA,
        pltpu.SemaphoreType.DMA,
        pltpu.SemaphoreType.DMA,
        pltpu.SemaphoreType.DMA,
        pltpu.SemaphoreType.DMA,
        pltpu.SemaphoreType.DMA,
        pltpu.SemaphoreType.DMA,
        pltpu.SemaphoreType.DMA,
    ],
    compiler_params=pltpu.CompilerParams(
        use_tc_tiling_on_sc=True, needs_layout_passes=False
    ),
)
def _emb_lookup(xt_hbm, t2_hbm, out_hbm,
                lo_v, hi_v, idx_a, idx_b, v0, v1, v2, v3, xwin,
                s_lo, s_hi, s_xa, s_xb, s_o0, s_o1, s_o2, s_o3, s_st):
    sid = lax.axis_index("s")
    wid = sid * 2 + lax.axis_index("c")
    idx_bufs = ((idx_a, s_xa), (idx_b, s_xb))
    val_bufs = ((v0, s_o0), (v1, s_o1), (v2, s_o2), (v3, s_o3))

    def pass0(ib, vb):
        def body(i, carry):
            base = i * (16 * _UNROLL)
            for u in range(_UNROLL):
                sl = pl.ds(base + u * 16, 16)
                vb[sl] = plsc.load_gather(
                    lo_v, [jnp.minimum(ib[sl], _LO - 1)])
            return carry

        lax.fori_loop(0, _NIT, body, 0)

    def pass1(ib, vb):
        def body(i, carry):
            base = i * (16 * _UNROLL)
            for u in range(_UNROLL):
                sl = pl.ds(base + u * 16, 16)
                idx = ib[sl]
                a = idx - _LO
                g = plsc.load_gather(hi_v, [jnp.maximum(a, 0)])
                vb[sl] = jnp.where(a >= 0, g, vb[sl])
            return carry

        lax.fori_loop(0, _NIT, body, 0)

    def x_issue(slot, chunk, par):
        buf, sem = idx_bufs[par % 2]
        pltpu.async_copy(xwin.at[slot, pl.ds(chunk * _BCH, _BCH)], buf, sem)

    def x_wait(slot, chunk, par):
        buf, sem = idx_bufs[par % 2]
        pltpu.make_async_copy(
            xwin.at[slot, pl.ds(chunk * _BCH, _BCH)], buf, sem).wait()
        return buf

    pltpu.async_copy(t2_hbm.at[wid, pl.ds(0, _LO)], lo_v, s_lo)
    pltpu.async_copy(t2_hbm.at[wid, pl.ds(_LO, _HI)], hi_v, s_hi)

    @pl.when(sid == 0)
    def _():
        pltpu.sync_copy(xt_hbm.at[0], xwin.at[0])

    plsc.subcore_barrier()
    x_issue(0, 0, 0)

    def field_body(j, carry):
        p = j * _EMBED_DIM + wid
        slot = j % 2
        nslot = (j + 1) % 2
        not_last = j + 1 < _NUM_FIELDS

        @pl.when((sid == 0) & not_last)
        def _():
            pltpu.async_copy(xt_hbm.at[j + 1], xwin.at[nslot], s_st)

        pltpu.make_async_copy(
            t2_hbm.at[p, pl.ds(0, _LO)], lo_v, s_lo).wait()
        for c in range(_NCH):
            x_issue(slot, (c + 1) % _NCH, c + 1)
            ib = x_wait(slot, c, c)
            vb, s_v = val_bufs[c]

            @pl.when(j > 0)
            def _():
                pltpu.make_async_copy(
                    vb, out_hbm.at[p - _EMBED_DIM, pl.ds(c * _BCH, _BCH)],
                    s_v).wait()

            pass0(ib, vb)

        @pl.when(not_last)
        def _():
            pltpu.async_copy(
                t2_hbm.at[p + _EMBED_DIM, pl.ds(0, _LO)], lo_v, s_lo)

        pltpu.make_async_copy(
            t2_hbm.at[p, pl.ds(_LO, _HI)], hi_v, s_hi).wait()
        for c in range(_NCH):
            if c + 1 < _NCH:
                x_issue(slot, c + 1, 5 + c)
            ib = x_wait(slot, c, 4 + c)
            vb, s_v = val_bufs[c]
            pass1(ib, vb)
            pltpu.async_copy(
                vb, out_hbm.at[p, pl.ds(c * _BCH, _BCH)], s_v)

        @pl.when(not_last)
        def _():
            pltpu.async_copy(
                t2_hbm.at[p + _EMBED_DIM, pl.ds(_LO, _HI)], hi_v, s_hi)

        @pl.when((sid == 0) & not_last)
        def _():
            pltpu.make_async_copy(xt_hbm.at[j + 1], xwin.at[nslot], s_st).wait()

        plsc.subcore_barrier()

        @pl.when(not_last)
        def _():
            x_issue(nslot, 0, 0)

        return carry

    lax.fori_loop(0, _NUM_FIELDS, field_body, 0)

    p_last = (_NUM_FIELDS - 1) * _EMBED_DIM + wid
    for c in range(_NCH):
        vb, s_v = val_bufs[c]
        pltpu.make_async_copy(
            vb, out_hbm.at[p_last, pl.ds(c * _BCH, _BCH)], s_v).wait()


def kernel(x, tables):
    xt = x.astype(jnp.int32).T
    t2 = tables.transpose(0, 2, 1).reshape(_NPLANE, _VOCAB)
    out = _emb_lookup(xt, t2)
    return out.reshape(_NUM_FIELDS, _EMBED_DIM, _BATCH).transpose(2, 0, 1)


# speedup vs baseline: 1.1921x; 1.1921x over previous
import functools

import jax
import jax.numpy as jnp
from jax import lax
from jax.experimental import pallas as pl
from jax.experimental.pallas import tpu as pltpu
from jax.experimental.pallas import tpu_sc as plsc

_NUM_FIELDS = 26
_VOCAB = 100000
_EMBED_DIM = 32
_BATCH = 16384
_NPLANE = _NUM_FIELDS * _EMBED_DIM
_BCH = 4096
_NCH = _BATCH // _BCH
_UNROLL = 8
_NIT = _BCH // (16 * _UNROLL)

_mesh = plsc.VectorSubcoreMesh(core_axis_name="c", subcore_axis_name="s")


@functools.partial(
    pl.kernel,
    mesh=_mesh,
    out_type=jax.ShapeDtypeStruct((_NPLANE, _BATCH), jnp.float32),
    scratch_types=[
        pltpu.VMEM((_VOCAB,), jnp.float32),
        pltpu.VMEM((_BCH,), jnp.int32),
        pltpu.VMEM((_BCH,), jnp.int32),
        pltpu.VMEM((_BCH,), jnp.float32),
        pltpu.VMEM((_BCH,), jnp.float32),
        pltpu.VMEM((_BCH,), jnp.float32),
        pltpu.VMEM((_BCH,), jnp.float32),
        pltpu.VMEM_SHARED((2, _BATCH), jnp.int32),
        pltpu.SemaphoreType.DMA,
        pltpu.SemaphoreType.DMA,
        pltpu.SemaphoreType.DMA,
        pltpu.SemaphoreType.DMA,
        pltpu.SemaphoreType.DMA,
        pltpu.SemaphoreType.DMA,
        pltpu.SemaphoreType.DMA,
        pltpu.SemaphoreType.DMA,
    ],
    compiler_params=pltpu.CompilerParams(
        use_tc_tiling_on_sc=True, needs_layout_passes=False
    ),
)
def _emb_lookup(xt_hbm, t2_hbm, out_hbm,
                row_v, idx_a, idx_b, v0, v1, v2, v3, xwin,
                s_row, s_xa, s_xb, s_o0, s_o1, s_o2, s_o3, s_st):
    sid = lax.axis_index("s")
    wid = sid * 2 + lax.axis_index("c")
    idx_bufs = ((idx_a, s_xa), (idx_b, s_xb))
    val_bufs = ((v0, s_o0), (v1, s_o1), (v2, s_o2), (v3, s_o3))

    def gather_chunk(ib, vb):
        def body(i, carry):
            base = i * (16 * _UNROLL)
            for u in range(_UNROLL):
                sl = pl.ds(base + u * 16, 16)
                vb[sl] = plsc.load_gather(row_v, [ib[sl]])
            return carry

        lax.fori_loop(0, _NIT, body, 0)

    h_row = pltpu.async_copy(t2_hbm.at[wid], row_v, s_row)

    @pl.when(sid == 0)
    def _():
        pltpu.sync_copy(xt_hbm.at[0], xwin.at[0])

    plsc.subcore_barrier()
    h_x = pltpu.async_copy(xwin.at[0, pl.ds(0, _BCH)], idx_a, s_xa)
    out_h = [None, None, None, None]

    for j in range(_NUM_FIELDS):
        p = j * _EMBED_DIM + wid
        if j + 1 < _NUM_FIELDS:

            @pl.when(sid == 0)
            def _():
                pltpu.async_copy(xt_hbm.at[j + 1], xwin.at[(j + 1) % 2], s_st)

        h_row.wait()
        for c in range(_NCH):
            ib, _ = idx_bufs[c % 2]
            vb, s_v = val_bufs[c]
            h_x.wait()
            if c + 1 < _NCH:
                nib, ns = idx_bufs[(c + 1) % 2]
                h_x = pltpu.async_copy(
                    xwin.at[j % 2, pl.ds((c + 1) * _BCH, _BCH)], nib, ns)
            if out_h[c] is not None:
                out_h[c].wait()
            gather_chunk(ib, vb)
            out_h[c] = pltpu.async_copy(
                vb, out_hbm.at[p, pl.ds(c * _BCH, _BCH)], s_v)
        if j + 1 < _NUM_FIELDS:
            h_row = pltpu.async_copy(
                t2_hbm.at[(j + 1) * _EMBED_DIM + wid], row_v, s_row)

            @pl.when(sid == 0)
            def _():
                pltpu.make_async_copy(
                    xt_hbm.at[j + 1], xwin.at[(j + 1) % 2], s_st).wait()

            plsc.subcore_barrier()
            nib, ns = idx_bufs[0]
            h_x = pltpu.async_copy(
                xwin.at[(j + 1) % 2, pl.ds(0, _BCH)], nib, ns)

    for h in out_h:
        h.wait()


def kernel(x, tables):
    xt = x.astype(jnp.int32).T
    t2 = tables.transpose(0, 2, 1).reshape(_NPLANE, _VOCAB)
    out = _emb_lookup(xt, t2)
    return out.reshape(_NUM_FIELDS, _EMBED_DIM, _BATCH).transpose(2, 0, 1)
